# baseline (speedup 1.0000x reference)
import functools

import jax
import jax.numpy as jnp
from jax import lax
from jax.experimental import pallas as pl
from jax.experimental.pallas import tpu as pltpu
from jax.experimental.pallas import tpu_sc as plsc

NC = 80
NA = 3
IMG = 640.0
CH = NA * (5 + NC)
HWS = ((80, 80), (40, 40), (20, 20))
B = 8
NB = 32
N = B * NB
SC_CORES = 2
SC_SUBCORES = 16
NW = SC_CORES * SC_SUBCORES
BPT = N // NW
CPB = 256
WPT = BPT * CPB
NDMA = WPT // 128


def _sc_body(p3, p4, p5, boxes, out, bx_v, r3, r4, r5, sem):
    cid = lax.axis_index("c")
    sid = lax.axis_index("s")
    wid = sid * SC_CORES + cid
    pltpu.sync_copy(boxes.at[pl.ds(wid * (BPT * 4), BPT * 4)], bx_v)
    lanes = lax.iota(jnp.int32, 16)
    nc4 = jnp.minimum(lanes, BPT - 1) * 4
    x1 = plsc.load_gather(bx_v, [nc4])
    y1 = plsc.load_gather(bx_v, [nc4 + 1])
    x2 = plsc.load_gather(bx_v, [nc4 + 2])
    y2 = plsc.load_gather(bx_v, [nc4 + 3])
    cx = jnp.clip((x1 + x2) * 0.5 / IMG, 0.0, 1.0 - 1e-6)
    cy = jnp.clip((y1 + y2) * 0.5 / IMG, 0.0, 1.0 - 1e-6)
    bimg = wid // (NB // BPT)

    rows = (r3, r4, r5)
    cps = []
    for s, (h, w) in enumerate(HWS):
        gi = jnp.clip((cx * float(w)).astype(jnp.int32), 0, w - 1)
        gj = jnp.clip((cy * float(h)).astype(jnp.int32), 0, h - 1)
        for m in range(BPT):
            gim = lax.reduce_max(jnp.where(lanes == m, gi, 0), (0,))
            gjm = lax.reduce_max(jnp.where(lanes == m, gj, 0), (0,))
            if s == 2:
                src = p5.at[gjm, gim, bimg, :]
            else:
                src = (p3, p4)[s].at[bimg, gjm, gim, :]
            cps.append(pltpu.make_async_copy(src, rows[s].at[m], sem))
    for c in cps:
        c.start()
    for c in cps:
        c.wait()
    for s in range(3):
        pltpu.sync_copy(rows[s], out.at[s, wid])


@functools.cache
def _sc_gather():
    return pl.kernel(
        _sc_body,
        out_type=jax.ShapeDtypeStruct((3, NW, BPT, CH), jnp.float32),
        mesh=plsc.VectorSubcoreMesh(
            core_axis_name="c", subcore_axis_name="s",
            num_cores=SC_CORES, num_subcores=SC_SUBCORES),
        compiler_params=pltpu.CompilerParams(needs_layout_passes=False),
        scratch_types=[
            pltpu.VMEM((BPT * 4,), jnp.float32),
            pltpu.VMEM((BPT, CH), jnp.float32),
            pltpu.VMEM((BPT, CH), jnp.float32),
            pltpu.VMEM((BPT, CH), jnp.float32),
            pltpu.SemaphoreType.DMA,
        ],
    )


def _bce(x, t):
    return jnp.maximum(x, 0.0) - x * t + jnp.log1p(jnp.exp(-jnp.abs(x)))


def _sig(x):
    return 1.0 / (1.0 + jnp.exp(-x))


def _meta(x1, y1, x2, y2):
    bw = jnp.clip((x2 - x1) / IMG, 1e-6, 1.0)
    bh = jnp.clip((y2 - y1) / IMG, 1e-6, 1.0)
    ms = jnp.maximum(bw, bh)
    s = jnp.where(ms < 0.15, 0, jnp.where(ms < 0.45, 1, 2))
    cx = jnp.clip((x1 + x2) * 0.5 / IMG, 0.0, 1.0 - 1e-6)
    cy = jnp.clip((y1 + y2) * 0.5 / IMG, 0.0, 1.0 - 1e-6)
    gis, gjs = [], []
    for (h, w) in HWS:
        gis.append(jnp.clip(jnp.floor(cx * w).astype(jnp.int32), 0, w - 1))
        gjs.append(jnp.clip(jnp.floor(cy * h).astype(jnp.int32), 0, h - 1))
    gi = jnp.where(s == 0, gis[0], jnp.where(s == 1, gis[1], gis[2]))
    gj = jnp.where(s == 0, gjs[0], jnp.where(s == 1, gjs[1], gjs[2]))
    return bw, bh, cx, cy, s, gi, gj


def _tc_body(o3_ref, o4_ref, o5_ref, g_ref, bx_ref, bxt_ref, lab_ref,
             labt_ref, out_ref):
    def fneg_sum(x):
        p = _sig(x)
        return jnp.sum(0.75 * p * p * _bce(x, 0.0))

    acc = [fneg_sum(o3_ref[...]), fneg_sum(o4_ref[...]), fneg_sum(o5_ref[...])]

    if True:
        boxes = bx_ref[...]
        bT = bxt_ref[...]
        lab = lab_ref[...]
        labT = labt_ref[...]

        bw, bh, cx, cy, s_c, gi_c, gj_c = _meta(
            boxes[:, 0:1], boxes[:, 1:2], boxes[:, 2:3], boxes[:, 3:4])
        _, _, _, _, s_r, gi_r, gj_r = _meta(
            bT[0:1, :], bT[1:2, :], bT[2:3, :], bT[3:4, :])

        bidx_c = lax.broadcasted_iota(jnp.int32, (N, 1), 0) // NB
        bidx_r = lax.broadcasted_iota(jnp.int32, (1, N), 1) // NB
        labc_c = jnp.clip(lab, 0, NC - 1)
        labc_r = jnp.clip(labT, 0, NC - 1)
        valid_c = (lab >= 0) & (lab < NC)
        valid_r = (labT >= 0) & (labT < NC)

        key_c = ((bidx_c * 4 + s_c) * 128 + gj_c) * 128 + gi_c
        key_r = ((bidx_r * 4 + s_r) * 128 + gj_r) * 128 + gi_r
        key2_c = key_c * 128 + labc_c
        key2_r = key_r * 128 + labc_r

        nm_lt = (lax.broadcasted_iota(jnp.int32, (N, N), 1)
                 < lax.broadcasted_iota(jnp.int32, (N, N), 0))
        occ = (key_c == key_r) & valid_r & nm_lt
        fc = valid_c & (jnp.max(occ.astype(jnp.int32), axis=1,
                                keepdims=True) == 0)
        occ2 = (key2_c == key2_r) & valid_r & nm_lt
        fcl = valid_c & (jnp.max(occ2.astype(jnp.int32), axis=1,
                                 keepdims=True) == 0)
        fc_f = fc.astype(jnp.float32)
        fcl_f = fcl.astype(jnp.float32)
        valid_f = valid_c.astype(jnp.float32)

        sel = [(s_c == s).astype(jnp.float32) for s in range(3)]
        g = g_ref[...]
        own = g[0] * sel[0] + g[1] * sel[1] + g[2] * sel[2]

        onehot = (labc_c == lax.broadcasted_iota(jnp.int32, (N, NC), 1)
                  ).astype(jnp.float32)

        corr_col = jnp.zeros((N, 1), jnp.float32)
        s0_col = jnp.zeros((N, 1), jnp.float32)
        dl_col = jnp.zeros((N, 1), jnp.float32)
        for an in range(NA):
            o = own[:, an * 85 + 4:an * 85 + 5]
            po = _sig(o)
            elem_pos = _bce(o, 1.0) * (0.25 * (1.0 - po) * (1.0 - po))
            elem_neg = _bce(o, 0.0) * (0.75 * po * po)
            corr_col += elem_pos - elem_neg
            cl = own[:, an * 85 + 5:an * 85 + 85]
            b0 = _bce(cl, 0.0)
            s0_col += jnp.sum(b0, axis=1, keepdims=True)
            dl_col += jnp.sum((_bce(cl, 1.0) - b0) * onehot, axis=1,
                              keepdims=True)
        corr_col = corr_col * fc_f
        cls_col = s0_col * fc_f + dl_col * fcl_f

        obj_loss = jnp.float32(0.0)
        cls_loss = jnp.float32(0.0)
        for s in range(3):
            pos = 3.0 * jnp.sum(fc_f * sel[s])
            denom = jnp.maximum(pos, 1.0)
            obj_loss += (acc[s] + jnp.sum(corr_col * sel[s])) / denom
            cls_loss += jnp.sum(cls_col * sel[s]) / jnp.maximum(pos * NC, 1.0)

        wv = sel[0] * 80.0 + sel[1] * 40.0 + sel[2] * 20.0
        hv = wv
        tx1 = cx - bw / 2
        ty1 = cy - bh / 2
        tx2 = cx + bw / 2
        ty2 = cy + bh / 2
        area2 = (tx2 - tx1) * (ty2 - ty1)
        gif = gi_c.astype(jnp.float32)
        gjf = gj_c.astype(jnp.float32)
        box_sum = jnp.float32(0.0)
        for an in range(NA):
            px = _sig(own[:, an * 85 + 0:an * 85 + 1])
            py = _sig(own[:, an * 85 + 1:an * 85 + 2])
            pw = _sig(own[:, an * 85 + 2:an * 85 + 3])
            ph = _sig(own[:, an * 85 + 3:an * 85 + 4])
            pcx = (gif + px) / wv
            pcy = (gjf + py) / hv
            px1 = pcx - pw / 2
            py1 = pcy - ph / 2
            px2 = pcx + pw / 2
            py2 = pcy + ph / 2
            area1 = (px2 - px1) * (py2 - py1)
            iw = jnp.maximum(jnp.minimum(px2, tx2) - jnp.maximum(px1, tx1), 0.0)
            ih = jnp.maximum(jnp.minimum(py2, ty2) - jnp.maximum(py1, ty1), 0.0)
            inter = iw * ih
            union = area1 + area2 - inter
            iou = inter / union
            cw = jnp.maximum(jnp.maximum(px2, tx2) - jnp.minimum(px1, tx1), 0.0)
            chh = jnp.maximum(jnp.maximum(py2, ty2) - jnp.minimum(py1, ty1), 0.0)
            areac = cw * chh
            gg = iou - (areac - union) / areac
            box_sum += jnp.sum((1.0 - gg) * valid_f)
        total_pos = 3.0 * jnp.sum(valid_f)
        loss = (obj_loss + 0.5 * cls_loss
                + 5.0 * box_sum / jnp.maximum(total_pos, 1.0))
        out_ref[...] = loss.reshape(1, 1)


def _combine_call(o3, o4, o5, g, bx, bxt, lab, labt):
    return pl.pallas_call(
        _tc_body,
        in_specs=[
            pl.BlockSpec((B, NA, 80, 80), lambda: (0, 0, 0, 0)),
            pl.BlockSpec((B, NA, 40, 40), lambda: (0, 0, 0, 0)),
            pl.BlockSpec((B, NA, 20, 20), lambda: (0, 0, 0, 0)),
            pl.BlockSpec((3, N, CH), lambda: (0, 0, 0)),
            pl.BlockSpec((N, 4), lambda: (0, 0)),
            pl.BlockSpec((4, N), lambda: (0, 0)),
            pl.BlockSpec((N, 1), lambda: (0, 0)),
            pl.BlockSpec((1, N), lambda: (0, 0)),
        ],
        out_specs=pl.BlockSpec((1, 1), lambda: (0, 0)),
        out_shape=jax.ShapeDtypeStruct((1, 1), jnp.float32),
    )(o3, o4, o5, g, bx, bxt, lab, labt)


def kernel(pred_p3, pred_p4, pred_p5, target_boxes, target_labels):
    pt3 = jnp.transpose(pred_p3, (0, 2, 3, 1))
    pt4 = jnp.transpose(pred_p4, (0, 2, 3, 1))
    pt5 = jnp.transpose(pred_p5, (2, 3, 0, 1))
    sc_out = _sc_gather()(pt3, pt4, pt5, target_boxes.reshape(-1))
    g = sc_out.reshape(3, N, CH)
    o3 = pred_p3[:, 4::85]
    o4 = pred_p4[:, 4::85]
    o5 = pred_p5[:, 4::85]
    bx = target_boxes.reshape(N, 4)
    lab32 = target_labels.astype(jnp.int32)
    out = _combine_call(
        o3, o4, o5, g, bx, bx.T,
        lab32.reshape(N, 1), lab32.reshape(1, N))
    return out[0, 0]


# speedup vs baseline: 1.1716x; 1.1716x over previous
import functools

import jax
import jax.numpy as jnp
from jax import lax
from jax.experimental import pallas as pl
from jax.experimental.pallas import tpu as pltpu
from jax.experimental.pallas import tpu_sc as plsc

NC = 80
NA = 3
IMG = 640.0
CH = NA * (5 + NC)
HWS = ((80, 80), (40, 40), (20, 20))
B = 8
NB = 32
N = B * NB
SC_CORES = 2
SC_SUBCORES = 16
NW = SC_CORES * SC_SUBCORES
BPT = N // NW
CPB = 256
WPT = BPT * CPB
NDMA = WPT // 128


def _sc_body(p3, p4, p5, boxes, out, bx_v, r3, r4, r5, sem):
    cid = lax.axis_index("c")
    sid = lax.axis_index("s")
    wid = sid * SC_CORES + cid
    pltpu.sync_copy(boxes.at[pl.ds(wid * (BPT * 4), BPT * 4)], bx_v)
    lanes = lax.iota(jnp.int32, 16)
    nc4 = jnp.minimum(lanes, BPT - 1) * 4
    x1 = plsc.load_gather(bx_v, [nc4])
    y1 = plsc.load_gather(bx_v, [nc4 + 1])
    x2 = plsc.load_gather(bx_v, [nc4 + 2])
    y2 = plsc.load_gather(bx_v, [nc4 + 3])
    cx = jnp.clip((x1 + x2) * 0.5 / IMG, 0.0, 1.0 - 1e-6)
    cy = jnp.clip((y1 + y2) * 0.5 / IMG, 0.0, 1.0 - 1e-6)
    bimg = wid // (NB // BPT)

    rows = (r3, r4, r5)
    cps = []
    for s, (h, w) in enumerate(HWS):
        gi = jnp.clip((cx * float(w)).astype(jnp.int32), 0, w - 1)
        gj = jnp.clip((cy * float(h)).astype(jnp.int32), 0, h - 1)
        for m in range(BPT):
            gim = lax.reduce_max(jnp.where(lanes == m, gi, 0), (0,))
            gjm = lax.reduce_max(jnp.where(lanes == m, gj, 0), (0,))
            if s == 2:
                src = p5.at[gjm, gim, bimg, :]
            else:
                src = (p3, p4)[s].at[bimg, gjm, gim, :]
            cps.append(pltpu.make_async_copy(src, rows[s].at[m], sem))
    for c in cps:
        c.start()
    for c in cps:
        c.wait()
    for s in range(3):
        pltpu.sync_copy(rows[s], out.at[s, wid])


@functools.cache
def _sc_gather():
    return pl.kernel(
        _sc_body,
        out_type=jax.ShapeDtypeStruct((3, NW, BPT, CH), jnp.float32),
        mesh=plsc.VectorSubcoreMesh(
            core_axis_name="c", subcore_axis_name="s",
            num_cores=SC_CORES, num_subcores=SC_SUBCORES),
        compiler_params=pltpu.CompilerParams(needs_layout_passes=False),
        scratch_types=[
            pltpu.VMEM((BPT * 4,), jnp.float32),
            pltpu.VMEM((BPT, CH), jnp.float32),
            pltpu.VMEM((BPT, CH), jnp.float32),
            pltpu.VMEM((BPT, CH), jnp.float32),
            pltpu.SemaphoreType.DMA,
        ],
    )


def _bce(x, t):
    return jnp.maximum(x, 0.0) - x * t + jnp.log1p(jnp.exp(-jnp.abs(x)))


def _sig(x):
    return 1.0 / (1.0 + jnp.exp(-x))


def _meta(x1, y1, x2, y2):
    bw = jnp.clip((x2 - x1) / IMG, 1e-6, 1.0)
    bh = jnp.clip((y2 - y1) / IMG, 1e-6, 1.0)
    ms = jnp.maximum(bw, bh)
    s = jnp.where(ms < 0.15, 0, jnp.where(ms < 0.45, 1, 2))
    cx = jnp.clip((x1 + x2) * 0.5 / IMG, 0.0, 1.0 - 1e-6)
    cy = jnp.clip((y1 + y2) * 0.5 / IMG, 0.0, 1.0 - 1e-6)
    gis, gjs = [], []
    for (h, w) in HWS:
        gis.append(jnp.clip(jnp.floor(cx * w).astype(jnp.int32), 0, w - 1))
        gjs.append(jnp.clip(jnp.floor(cy * h).astype(jnp.int32), 0, h - 1))
    gi = jnp.where(s == 0, gis[0], jnp.where(s == 1, gis[1], gis[2]))
    gj = jnp.where(s == 0, gjs[0], jnp.where(s == 1, gjs[1], gjs[2]))
    return bw, bh, cx, cy, s, gi, gj


RCH = 3200
GSTEPS = B * 80 * 80 // RCH


def _tc_body(p3_ref, p4_ref, p5_ref, g_ref, bx_ref, bxt_ref, lab_ref,
             labt_ref, out_ref, acc):
    a = pl.program_id(0)

    @pl.when(a == 0)
    def _init():
        acc[0] = 0.0
        acc[1] = 0.0
        acc[2] = 0.0

    def fneg_sum(blk):
        tot = jnp.float32(0.0)
        for an in range(NA):
            x = blk[:, 4 + 85 * an:5 + 85 * an]
            p = _sig(x)
            tot += jnp.sum(0.75 * p * p * _bce(x, 0.0))
        return tot

    acc[0] += fneg_sum(p3_ref[...])
    acc[1] += jnp.where(a < 4, fneg_sum(p4_ref[...]), 0.0)
    acc[2] += jnp.where(a == 0, fneg_sum(p5_ref[...]), 0.0)

    @pl.when(a == GSTEPS - 1)
    def _run_combine():
        _combine_math(g_ref, bx_ref, bxt_ref, lab_ref, labt_ref, out_ref, acc)


def _combine_math(g_ref, bx_ref, bxt_ref, lab_ref, labt_ref, out_ref, acc):
    if True:
        boxes = bx_ref[...]
        bT = bxt_ref[...]
        lab = lab_ref[...]
        labT = labt_ref[...]

        bw, bh, cx, cy, s_c, gi_c, gj_c = _meta(
            boxes[:, 0:1], boxes[:, 1:2], boxes[:, 2:3], boxes[:, 3:4])
        _, _, _, _, s_r, gi_r, gj_r = _meta(
            bT[0:1, :], bT[1:2, :], bT[2:3, :], bT[3:4, :])

        bidx_c = lax.broadcasted_iota(jnp.int32, (N, 1), 0) // NB
        bidx_r = lax.broadcasted_iota(jnp.int32, (1, N), 1) // NB
        labc_c = jnp.clip(lab, 0, NC - 1)
        labc_r = jnp.clip(labT, 0, NC - 1)
        valid_c = (lab >= 0) & (lab < NC)
        valid_r = (labT >= 0) & (labT < NC)

        key_c = ((bidx_c * 4 + s_c) * 128 + gj_c) * 128 + gi_c
        key_r = ((bidx_r * 4 + s_r) * 128 + gj_r) * 128 + gi_r
        key2_c = key_c * 128 + labc_c
        key2_r = key_r * 128 + labc_r

        nm_lt = (lax.broadcasted_iota(jnp.int32, (N, N), 1)
                 < lax.broadcasted_iota(jnp.int32, (N, N), 0))
        occ = (key_c == key_r) & valid_r & nm_lt
        fc = valid_c & (jnp.max(occ.astype(jnp.int32), axis=1,
                                keepdims=True) == 0)
        occ2 = (key2_c == key2_r) & valid_r & nm_lt
        fcl = valid_c & (jnp.max(occ2.astype(jnp.int32), axis=1,
                                 keepdims=True) == 0)
        fc_f = fc.astype(jnp.float32)
        fcl_f = fcl.astype(jnp.float32)
        valid_f = valid_c.astype(jnp.float32)

        sel = [(s_c == s).astype(jnp.float32) for s in range(3)]
        g = g_ref[...]
        own = g[0] * sel[0] + g[1] * sel[1] + g[2] * sel[2]

        onehot = (labc_c == lax.broadcasted_iota(jnp.int32, (N, NC), 1)
                  ).astype(jnp.float32)

        corr_col = jnp.zeros((N, 1), jnp.float32)
        s0_col = jnp.zeros((N, 1), jnp.float32)
        dl_col = jnp.zeros((N, 1), jnp.float32)
        for an in range(NA):
            o = own[:, an * 85 + 4:an * 85 + 5]
            po = _sig(o)
            elem_pos = _bce(o, 1.0) * (0.25 * (1.0 - po) * (1.0 - po))
            elem_neg = _bce(o, 0.0) * (0.75 * po * po)
            corr_col += elem_pos - elem_neg
            cl = own[:, an * 85 + 5:an * 85 + 85]
            b0 = _bce(cl, 0.0)
            s0_col += jnp.sum(b0, axis=1, keepdims=True)
            dl_col += jnp.sum((_bce(cl, 1.0) - b0) * onehot, axis=1,
                              keepdims=True)
        corr_col = corr_col * fc_f
        cls_col = s0_col * fc_f + dl_col * fcl_f

        obj_loss = jnp.float32(0.0)
        cls_loss = jnp.float32(0.0)
        for s in range(3):
            pos = 3.0 * jnp.sum(fc_f * sel[s])
            denom = jnp.maximum(pos, 1.0)
            obj_loss += (acc[s] + jnp.sum(corr_col * sel[s])) / denom
            cls_loss += jnp.sum(cls_col * sel[s]) / jnp.maximum(pos * NC, 1.0)

        wv = sel[0] * 80.0 + sel[1] * 40.0 + sel[2] * 20.0
        hv = wv
        tx1 = cx - bw / 2
        ty1 = cy - bh / 2
        tx2 = cx + bw / 2
        ty2 = cy + bh / 2
        area2 = (tx2 - tx1) * (ty2 - ty1)
        gif = gi_c.astype(jnp.float32)
        gjf = gj_c.astype(jnp.float32)
        box_sum = jnp.float32(0.0)
        for an in range(NA):
            px = _sig(own[:, an * 85 + 0:an * 85 + 1])
            py = _sig(own[:, an * 85 + 1:an * 85 + 2])
            pw = _sig(own[:, an * 85 + 2:an * 85 + 3])
            ph = _sig(own[:, an * 85 + 3:an * 85 + 4])
            pcx = (gif + px) / wv
            pcy = (gjf + py) / hv
            px1 = pcx - pw / 2
            py1 = pcy - ph / 2
            px2 = pcx + pw / 2
            py2 = pcy + ph / 2
            area1 = (px2 - px1) * (py2 - py1)
            iw = jnp.maximum(jnp.minimum(px2, tx2) - jnp.maximum(px1, tx1), 0.0)
            ih = jnp.maximum(jnp.minimum(py2, ty2) - jnp.maximum(py1, ty1), 0.0)
            inter = iw * ih
            union = area1 + area2 - inter
            iou = inter / union
            cw = jnp.maximum(jnp.maximum(px2, tx2) - jnp.minimum(px1, tx1), 0.0)
            chh = jnp.maximum(jnp.maximum(py2, ty2) - jnp.minimum(py1, ty1), 0.0)
            areac = cw * chh
            gg = iou - (areac - union) / areac
            box_sum += jnp.sum((1.0 - gg) * valid_f)
        total_pos = 3.0 * jnp.sum(valid_f)
        loss = (obj_loss + 0.5 * cls_loss
                + 5.0 * box_sum / jnp.maximum(total_pos, 1.0))
        out_ref[...] = loss.reshape(1, 1)


def _combine_call(p3r, p4r, p5r, g, bx, bxt, lab, labt):
    return pl.pallas_call(
        _tc_body,
        grid=(GSTEPS,),
        in_specs=[
            pl.BlockSpec((RCH, CH), lambda a: (a, 0)),
            pl.BlockSpec((RCH, CH), lambda a: (jnp.minimum(a, 3), 0)),
            pl.BlockSpec((RCH, CH), lambda a: (0, 0)),
            pl.BlockSpec((3, N, CH), lambda a: (0, 0, 0)),
            pl.BlockSpec((N, 4), lambda a: (0, 0)),
            pl.BlockSpec((4, N), lambda a: (0, 0)),
            pl.BlockSpec((N, 1), lambda a: (0, 0)),
            pl.BlockSpec((1, N), lambda a: (0, 0)),
        ],
        out_specs=pl.BlockSpec((1, 1), lambda a: (0, 0)),
        out_shape=jax.ShapeDtypeStruct((1, 1), jnp.float32),
        scratch_shapes=[pltpu.SMEM((4,), jnp.float32)],
    )(p3r, p4r, p5r, g, bx, bxt, lab, labt)


def kernel(pred_p3, pred_p4, pred_p5, target_boxes, target_labels):
    pt3 = jnp.transpose(pred_p3, (0, 2, 3, 1))
    pt4 = jnp.transpose(pred_p4, (0, 2, 3, 1))
    pt5 = jnp.transpose(pred_p5, (2, 3, 0, 1))
    sc_out = _sc_gather()(pt3, pt4, pt5, target_boxes.reshape(-1))
    g = sc_out.reshape(3, N, CH)
    bx = target_boxes.reshape(N, 4)
    lab32 = target_labels.astype(jnp.int32)
    out = _combine_call(
        pt3.reshape(B * 80 * 80, CH), pt4.reshape(B * 40 * 40, CH),
        pt5.reshape(20 * 20 * B, CH), g, bx, bx.T,
        lab32.reshape(N, 1), lab32.reshape(1, N))
    return out[0, 0]


# speedup vs baseline: 2.1931x; 1.8719x over previous
import functools

import jax
import jax.numpy as jnp
from jax import lax
from jax.experimental import pallas as pl
from jax.experimental.pallas import tpu as pltpu
from jax.experimental.pallas import tpu_sc as plsc

NC = 80
NA = 3
IMG = 640.0
CH = NA * (5 + NC)
HWS = ((80, 80), (40, 40), (20, 20))
B = 8
NB = 32
N = B * NB
SC_CORES = 2
SC_SUBCORES = 16
NW = SC_CORES * SC_SUBCORES
BPT = N // NW
CPB = 256
WPT = BPT * CPB
NDMA = WPT // 128


def _sc_body(p3, p4, p5, boxes, out, bx_v, r3, r4, r5, sem):
    cid = lax.axis_index("c")
    sid = lax.axis_index("s")
    wid = sid * SC_CORES + cid
    pltpu.sync_copy(boxes.at[pl.ds(wid * (BPT * 4), BPT * 4)], bx_v)
    lanes = lax.iota(jnp.int32, 16)
    nc4 = jnp.minimum(lanes, BPT - 1) * 4
    x1 = plsc.load_gather(bx_v, [nc4])
    y1 = plsc.load_gather(bx_v, [nc4 + 1])
    x2 = plsc.load_gather(bx_v, [nc4 + 2])
    y2 = plsc.load_gather(bx_v, [nc4 + 3])
    cx = jnp.clip((x1 + x2) * 0.5 / IMG, 0.0, 1.0 - 1e-6)
    cy = jnp.clip((y1 + y2) * 0.5 / IMG, 0.0, 1.0 - 1e-6)
    bimg = wid // (NB // BPT)

    rows = (r3, r4, r5)
    cps = []
    for s, (h, w) in enumerate(HWS):
        gi = jnp.clip((cx * float(w)).astype(jnp.int32), 0, w - 1)
        gj = jnp.clip((cy * float(h)).astype(jnp.int32), 0, h - 1)
        for m in range(BPT):
            gim = lax.reduce_max(jnp.where(lanes == m, gi, 0), (0,))
            gjm = lax.reduce_max(jnp.where(lanes == m, gj, 0), (0,))
            if s == 2:
                src = p5.at[gjm, gim, bimg, :]
            else:
                src = (p3, p4)[s].at[bimg, gjm, gim, :]
            cps.append(pltpu.make_async_copy(src, rows[s].at[m], sem))
    for c in cps:
        c.start()
    for c in cps:
        c.wait()
    for s in range(3):
        pltpu.sync_copy(rows[s], out.at[s, wid])


@functools.cache
def _sc_gather():
    return pl.kernel(
        _sc_body,
        out_type=jax.ShapeDtypeStruct((3, NW, BPT, CH), jnp.float32),
        mesh=plsc.VectorSubcoreMesh(
            core_axis_name="c", subcore_axis_name="s",
            num_cores=SC_CORES, num_subcores=SC_SUBCORES),
        compiler_params=pltpu.CompilerParams(needs_layout_passes=False),
        scratch_types=[
            pltpu.VMEM((BPT * 4,), jnp.float32),
            pltpu.VMEM((BPT, CH), jnp.float32),
            pltpu.VMEM((BPT, CH), jnp.float32),
            pltpu.VMEM((BPT, CH), jnp.float32),
            pltpu.SemaphoreType.DMA,
        ],
    )


def _bce(x, t):
    return jnp.maximum(x, 0.0) - x * t + jnp.log1p(jnp.exp(-jnp.abs(x)))


def _sig(x):
    return 1.0 / (1.0 + jnp.exp(-x))


def _meta(x1, y1, x2, y2):
    bw = jnp.clip((x2 - x1) / IMG, 1e-6, 1.0)
    bh = jnp.clip((y2 - y1) / IMG, 1e-6, 1.0)
    ms = jnp.maximum(bw, bh)
    s = jnp.where(ms < 0.15, 0, jnp.where(ms < 0.45, 1, 2))
    cx = jnp.clip((x1 + x2) * 0.5 / IMG, 0.0, 1.0 - 1e-6)
    cy = jnp.clip((y1 + y2) * 0.5 / IMG, 0.0, 1.0 - 1e-6)
    gis, gjs = [], []
    for (h, w) in HWS:
        gis.append(jnp.clip(jnp.floor(cx * w).astype(jnp.int32), 0, w - 1))
        gjs.append(jnp.clip(jnp.floor(cy * h).astype(jnp.int32), 0, h - 1))
    gi = jnp.where(s == 0, gis[0], jnp.where(s == 1, gis[1], gis[2]))
    gj = jnp.where(s == 0, gjs[0], jnp.where(s == 1, gjs[1], gjs[2]))
    return bw, bh, cx, cy, s, gi, gj


RCH = 3200
GSTEPS = B * 80 * 80 // RCH


def _tc_body(p3_ref, p4_ref, p5_ref, g_ref, bx_ref, bxt_ref, lab_ref,
             labt_ref, out_ref, acc):
    a = pl.program_id(0)

    @pl.when(a == 0)
    def _init():
        acc[0] = 0.0
        acc[1] = 0.0
        acc[2] = 0.0

    def fneg_sum(blk):
        tot = jnp.float32(0.0)
        for an in range(NA):
            x = blk[:, 4 + 85 * an:5 + 85 * an]
            p = _sig(x)
            tot += jnp.sum(0.75 * p * p * _bce(x, 0.0))
        return tot

    acc[0] += fneg_sum(p3_ref[...])

    @pl.when(a < 4)
    def _p4():
        acc[1] += fneg_sum(p4_ref[...])

    @pl.when(a == 0)
    def _p5():
        acc[2] += fneg_sum(p5_ref[...])

    @pl.when(a == GSTEPS - 1)
    def _run_combine():
        _combine_math(g_ref, bx_ref, bxt_ref, lab_ref, labt_ref, out_ref, acc)


def _combine_math(g_ref, bx_ref, bxt_ref, lab_ref, labt_ref, out_ref, acc):
    if True:
        boxes = bx_ref[...]
        bT = bxt_ref[...]
        lab = lab_ref[...]
        labT = labt_ref[...]

        bw, bh, cx, cy, s_c, gi_c, gj_c = _meta(
            boxes[:, 0:1], boxes[:, 1:2], boxes[:, 2:3], boxes[:, 3:4])
        _, _, _, _, s_r, gi_r, gj_r = _meta(
            bT[0:1, :], bT[1:2, :], bT[2:3, :], bT[3:4, :])

        bidx_c = lax.broadcasted_iota(jnp.int32, (N, 1), 0) // NB
        bidx_r = lax.broadcasted_iota(jnp.int32, (1, N), 1) // NB
        labc_c = jnp.clip(lab, 0, NC - 1)
        labc_r = jnp.clip(labT, 0, NC - 1)
        valid_c = (lab >= 0) & (lab < NC)
        valid_r = (labT >= 0) & (labT < NC)

        key_c = ((bidx_c * 4 + s_c) * 128 + gj_c) * 128 + gi_c
        key_r = ((bidx_r * 4 + s_r) * 128 + gj_r) * 128 + gi_r
        key2_c = key_c * 128 + labc_c
        key2_r = key_r * 128 + labc_r

        nm_lt = (lax.broadcasted_iota(jnp.int32, (N, N), 1)
                 < lax.broadcasted_iota(jnp.int32, (N, N), 0))
        occ = (key_c == key_r) & valid_r & nm_lt
        fc = valid_c & (jnp.max(occ.astype(jnp.int32), axis=1,
                                keepdims=True) == 0)
        occ2 = (key2_c == key2_r) & valid_r & nm_lt
        fcl = valid_c & (jnp.max(occ2.astype(jnp.int32), axis=1,
                                 keepdims=True) == 0)
        fc_f = fc.astype(jnp.float32)
        fcl_f = fcl.astype(jnp.float32)
        valid_f = valid_c.astype(jnp.float32)

        sel = [(s_c == s).astype(jnp.float32) for s in range(3)]
        g = g_ref[...]
        own = g[0] * sel[0] + g[1] * sel[1] + g[2] * sel[2]

        onehot = (labc_c == lax.broadcasted_iota(jnp.int32, (N, NC), 1)
                  ).astype(jnp.float32)

        corr_col = jnp.zeros((N, 1), jnp.float32)
        s0_col = jnp.zeros((N, 1), jnp.float32)
        dl_col = jnp.zeros((N, 1), jnp.float32)
        for an in range(NA):
            o = own[:, an * 85 + 4:an * 85 + 5]
            po = _sig(o)
            elem_pos = _bce(o, 1.0) * (0.25 * (1.0 - po) * (1.0 - po))
            elem_neg = _bce(o, 0.0) * (0.75 * po * po)
            corr_col += elem_pos - elem_neg
            cl = own[:, an * 85 + 5:an * 85 + 85]
            b0 = _bce(cl, 0.0)
            s0_col += jnp.sum(b0, axis=1, keepdims=True)
            dl_col += jnp.sum((_bce(cl, 1.0) - b0) * onehot, axis=1,
                              keepdims=True)
        corr_col = corr_col * fc_f
        cls_col = s0_col * fc_f + dl_col * fcl_f

        obj_loss = jnp.float32(0.0)
        cls_loss = jnp.float32(0.0)
        for s in range(3):
            pos = 3.0 * jnp.sum(fc_f * sel[s])
            denom = jnp.maximum(pos, 1.0)
            obj_loss += (acc[s] + jnp.sum(corr_col * sel[s])) / denom
            cls_loss += jnp.sum(cls_col * sel[s]) / jnp.maximum(pos * NC, 1.0)

        wv = sel[0] * 80.0 + sel[1] * 40.0 + sel[2] * 20.0
        hv = wv
        tx1 = cx - bw / 2
        ty1 = cy - bh / 2
        tx2 = cx + bw / 2
        ty2 = cy + bh / 2
        area2 = (tx2 - tx1) * (ty2 - ty1)
        gif = gi_c.astype(jnp.float32)
        gjf = gj_c.astype(jnp.float32)
        box_sum = jnp.float32(0.0)
        for an in range(NA):
            px = _sig(own[:, an * 85 + 0:an * 85 + 1])
            py = _sig(own[:, an * 85 + 1:an * 85 + 2])
            pw = _sig(own[:, an * 85 + 2:an * 85 + 3])
            ph = _sig(own[:, an * 85 + 3:an * 85 + 4])
            pcx = (gif + px) / wv
            pcy = (gjf + py) / hv
            px1 = pcx - pw / 2
            py1 = pcy - ph / 2
            px2 = pcx + pw / 2
            py2 = pcy + ph / 2
            area1 = (px2 - px1) * (py2 - py1)
            iw = jnp.maximum(jnp.minimum(px2, tx2) - jnp.maximum(px1, tx1), 0.0)
            ih = jnp.maximum(jnp.minimum(py2, ty2) - jnp.maximum(py1, ty1), 0.0)
            inter = iw * ih
            union = area1 + area2 - inter
            iou = inter / union
            cw = jnp.maximum(jnp.maximum(px2, tx2) - jnp.minimum(px1, tx1), 0.0)
            chh = jnp.maximum(jnp.maximum(py2, ty2) - jnp.minimum(py1, ty1), 0.0)
            areac = cw * chh
            gg = iou - (areac - union) / areac
            box_sum += jnp.sum((1.0 - gg) * valid_f)
        total_pos = 3.0 * jnp.sum(valid_f)
        loss = (obj_loss + 0.5 * cls_loss
                + 5.0 * box_sum / jnp.maximum(total_pos, 1.0))
        out_ref[...] = loss.reshape(1, 1)


def _combine_call(p3r, p4r, p5r, g, bx, bxt, lab, labt):
    return pl.pallas_call(
        _tc_body,
        grid=(GSTEPS,),
        in_specs=[
            pl.BlockSpec((RCH, CH), lambda a: (a, 0)),
            pl.BlockSpec((RCH, CH), lambda a: (jnp.minimum(a, 3), 0)),
            pl.BlockSpec((RCH, CH), lambda a: (0, 0)),
            pl.BlockSpec((3, N, CH), lambda a: (0, 0, 0)),
            pl.BlockSpec((N, 4), lambda a: (0, 0)),
            pl.BlockSpec((4, N), lambda a: (0, 0)),
            pl.BlockSpec((N, 1), lambda a: (0, 0)),
            pl.BlockSpec((1, N), lambda a: (0, 0)),
        ],
        out_specs=pl.BlockSpec((1, 1), lambda a: (0, 0)),
        out_shape=jax.ShapeDtypeStruct((1, 1), jnp.float32),
        scratch_shapes=[pltpu.SMEM((4,), jnp.float32)],
    )(p3r, p4r, p5r, g, bx, bxt, lab, labt)


def kernel(pred_p3, pred_p4, pred_p5, target_boxes, target_labels):
    pt3 = jnp.transpose(pred_p3, (0, 2, 3, 1))
    pt4 = jnp.transpose(pred_p4, (0, 2, 3, 1))
    pt5 = jnp.transpose(pred_p5, (2, 3, 0, 1))
    sc_out = _sc_gather()(pt3, pt4, pt5, target_boxes.reshape(-1))
    g = sc_out.reshape(3, N, CH)
    bx = target_boxes.reshape(N, 4)
    lab32 = target_labels.astype(jnp.int32)
    out = _combine_call(
        pt3.reshape(B * 80 * 80, CH), pt4.reshape(B * 40 * 40, CH),
        pt5.reshape(20 * 20 * B, CH), g, bx, bx.T,
        lab32.reshape(N, 1), lab32.reshape(1, N))
    return out[0, 0]
